# baseline (speedup 1.0000x reference)
import functools

import jax
import jax.numpy as jnp
from jax import lax
from jax.experimental import pallas as pl
from jax.experimental.pallas import tpu as pltpu
from jax.experimental.pallas import tpu_sc as plsc

NC = 2
NS = 16
L = 16
CH = 128


def _ee_body(bpw, d, p, uidx_hbm, iidx_hbm, eu_hbm, ei_hbm, bu_hbm, bi_hbm,
             gm_hbm, out_hbm, uidx_v, iidx_v, upk_v, ipk_v, urows_v, irows_v,
             ubias_v, ibias_v, gm_v, out_v, sem):
  wid = lax.axis_index("s") * NC + lax.axis_index("c")
  base = wid * bpw
  shift = p.bit_length() - 1
  dshift = d.bit_length() - 1

  pltpu.sync_copy(uidx_hbm.at[pl.ds(base, bpw)], uidx_v)
  pltpu.sync_copy(iidx_hbm.at[pl.ds(base, bpw)], iidx_v)
  pltpu.sync_copy(gm_hbm, gm_v)

  cb1 = pltpu.async_copy(bu_hbm.at[uidx_v], ubias_v, sem)
  cb2 = pltpu.async_copy(bi_hbm.at[iidx_v], ibias_v, sem)

  def packrow(j, carry):
    s = pl.ds(j * L, L)
    upk_v[s] = lax.shift_right_logical(uidx_v[s], shift)
    ipk_v[s] = lax.shift_right_logical(iidx_v[s], shift)
    return carry

  lax.fori_loop(0, bpw // L, packrow, 0)

  gm = gm_v[...]
  lane = lax.iota(jnp.int32, L)
  submask = jnp.full((L,), p - 1, jnp.int32)

  def chunk(k, carry):
    cbase = k * CH
    c1 = pltpu.async_copy(eu_hbm.at[upk_v.at[pl.ds(cbase, CH)]], urows_v, sem)
    c2 = pltpu.async_copy(ei_hbm.at[ipk_v.at[pl.ds(cbase, CH)]], irows_v, sem)
    c1.wait()
    c2.wait()

    def group(g, carry2):
      lbase = cbase + g * L
      uvec = uidx_v[pl.ds(lbase, L)]
      ivec = iidx_v[pl.ds(lbase, L)]
      ucol = lax.shift_left(uvec & submask, dshift)
      icol = lax.shift_left(ivec & submask, dshift)
      row = lane + g * L
      acc = jnp.zeros((L,), jnp.float32)
      for dd in range(d):
        uv = plsc.load_gather(urows_v, [row, ucol + dd])
        iv = plsc.load_gather(irows_v, [row, icol + dd])
        df = uv - iv
        acc = acc + df * df
      accs = jnp.maximum(acc, jnp.float32(1e-30))
      yi = jnp.int32(0x5F3759DF) - lax.shift_right_logical(
          plsc.bitcast(accs, jnp.int32), 1)
      y = plsc.bitcast(yi, jnp.float32)
      for _ in range(3):
        y = y * (jnp.float32(1.5) - jnp.float32(0.5) * accs * y * y)
      dist = acc * y
      ub = ubias_v[pl.ds(lbase, L)]
      ib = ibias_v[pl.ds(lbase, L)]
      x = gm + ub + ib - dist
      out_v[pl.ds(lbase, L)] = jnp.float32(1.0) / (jnp.float32(1.0) +
                                                   jnp.exp(-x))
      return carry2

    lax.fori_loop(0, CH // L, group, 0)
    return carry

  cb1.wait()
  cb2.wait()
  lax.fori_loop(0, bpw // CH, chunk, 0)
  pltpu.sync_copy(out_v, out_hbm.at[pl.ds(base, bpw)])


def kernel(user_indices, item_indices, embedding_user, embedding_item,
           bias_user, bias_item, global_mean=0.0):
  b = user_indices.shape[0]
  nu, d = embedding_user.shape
  ni = embedding_item.shape[0]
  p = 128 // d
  nw = NC * NS
  bpw = b // nw
  uidx = user_indices.astype(jnp.int32)
  iidx = item_indices.astype(jnp.int32)
  eu_pk = embedding_user.reshape(nu // p, d * p)
  ei_pk = embedding_item.reshape(ni // p, d * p)
  gm_arr = jnp.full((L,), global_mean, jnp.float32)

  mesh = plsc.VectorSubcoreMesh(core_axis_name="c", subcore_axis_name="s",
                                num_cores=NC, num_subcores=NS)
  run = pl.kernel(
      functools.partial(_ee_body, bpw, d, p),
      out_type=jax.ShapeDtypeStruct((b,), jnp.float32),
      mesh=mesh,
      compiler_params=pltpu.CompilerParams(needs_layout_passes=False),
      scratch_types=[
          pltpu.VMEM((bpw,), jnp.int32),
          pltpu.VMEM((bpw,), jnp.int32),
          pltpu.VMEM((bpw,), jnp.int32),
          pltpu.VMEM((bpw,), jnp.int32),
          pltpu.VMEM((CH, d * p), jnp.float32),
          pltpu.VMEM((CH, d * p), jnp.float32),
          pltpu.VMEM((bpw,), jnp.float32),
          pltpu.VMEM((bpw,), jnp.float32),
          pltpu.VMEM((L,), jnp.float32),
          pltpu.VMEM((bpw,), jnp.float32),
          pltpu.SemaphoreType.DMA,
      ],
  )
  return run(uidx, iidx, eu_pk, ei_pk,
             bias_user.astype(jnp.float32), bias_item.astype(jnp.float32),
             gm_arr)


# speedup vs baseline: 1.0099x; 1.0099x over previous
import functools

import jax
import jax.numpy as jnp
from jax import lax
from jax.experimental import pallas as pl
from jax.experimental.pallas import tpu as pltpu
from jax.experimental.pallas import tpu_sc as plsc

NC = 2
NS = 16
L = 16
CH = 128


def _ee_body(bpw, d, p, uidx_hbm, iidx_hbm, eu_hbm, ei_hbm, bu_hbm, bi_hbm,
             gm_hbm, out_hbm, uidx_v, iidx_v, upk_v, ipk_v, urows0_v,
             irows0_v, urows1_v, irows1_v, ubias_v, ibias_v, gm_v, out_v,
             bsem, sem0, sem1):
  wid = lax.axis_index("s") * NC + lax.axis_index("c")
  base = wid * bpw
  shift = p.bit_length() - 1
  dshift = d.bit_length() - 1
  nch = bpw // CH

  pltpu.sync_copy(uidx_hbm.at[pl.ds(base, bpw)], uidx_v)
  pltpu.sync_copy(iidx_hbm.at[pl.ds(base, bpw)], iidx_v)
  pltpu.sync_copy(gm_hbm, gm_v)

  cb1 = pltpu.async_copy(bu_hbm.at[uidx_v], ubias_v, bsem)
  cb2 = pltpu.async_copy(bi_hbm.at[iidx_v], ibias_v, bsem)

  def packrow(j, carry):
    s = pl.ds(j * L, L)
    upk_v[s] = lax.shift_right_logical(uidx_v[s], shift)
    ipk_v[s] = lax.shift_right_logical(iidx_v[s], shift)
    return carry

  lax.fori_loop(0, bpw // L, packrow, 0)

  gm = gm_v[...]
  lane = lax.iota(jnp.int32, L)
  submask = jnp.full((L,), p - 1, jnp.int32)
  bufs = ((urows0_v, irows0_v, sem0), (urows1_v, irows1_v, sem1))

  def fire(k):
    urows_v, irows_v, sem = bufs[k % 2]
    cbase = k * CH
    c1 = pltpu.async_copy(eu_hbm.at[upk_v.at[pl.ds(cbase, CH)]], urows_v, sem)
    c2 = pltpu.async_copy(ei_hbm.at[ipk_v.at[pl.ds(cbase, CH)]], irows_v, sem)
    return c1, c2

  def run_chunk(k):
    urows_v, irows_v, _ = bufs[k % 2]
    cbase = k * CH

    def group(g, carry2):
      lbase = cbase + g * L
      uvec = uidx_v[pl.ds(lbase, L)]
      ivec = iidx_v[pl.ds(lbase, L)]
      ucol = lax.shift_left(uvec & submask, dshift)
      icol = lax.shift_left(ivec & submask, dshift)
      row = lane + g * L
      acc = jnp.zeros((L,), jnp.float32)
      for dd in range(d):
        uv = plsc.load_gather(urows_v, [row, ucol + dd])
        iv = plsc.load_gather(irows_v, [row, icol + dd])
        df = uv - iv
        acc = acc + df * df
      accs = jnp.maximum(acc, jnp.float32(1e-30))
      yi = jnp.int32(0x5F3759DF) - lax.shift_right_logical(
          plsc.bitcast(accs, jnp.int32), 1)
      y = plsc.bitcast(yi, jnp.float32)
      for _ in range(3):
        y = y * (jnp.float32(1.5) - jnp.float32(0.5) * accs * y * y)
      dist = acc * y
      ub = ubias_v[pl.ds(lbase, L)]
      ib = ibias_v[pl.ds(lbase, L)]
      x = gm + ub + ib - dist
      out_v[pl.ds(lbase, L)] = jnp.float32(1.0) / (jnp.float32(1.0) +
                                                   jnp.exp(-x))
      return carry2

    lax.fori_loop(0, CH // L, group, 0)

  cb1.wait()
  cb2.wait()
  pending = fire(0)
  for k in range(nch):
    nxt = fire(k + 1) if k + 1 < nch else None
    pending[0].wait()
    pending[1].wait()
    run_chunk(k)
    pending = nxt
  pltpu.sync_copy(out_v, out_hbm.at[pl.ds(base, bpw)])


def kernel(user_indices, item_indices, embedding_user, embedding_item,
           bias_user, bias_item, global_mean=0.0):
  b = user_indices.shape[0]
  nu, d = embedding_user.shape
  ni = embedding_item.shape[0]
  p = 128 // d
  nw = NC * NS
  bpw = b // nw
  uidx = user_indices.astype(jnp.int32)
  iidx = item_indices.astype(jnp.int32)
  eu_pk = embedding_user.reshape(nu // p, d * p)
  ei_pk = embedding_item.reshape(ni // p, d * p)
  gm_arr = jnp.full((L,), global_mean, jnp.float32)

  mesh = plsc.VectorSubcoreMesh(core_axis_name="c", subcore_axis_name="s",
                                num_cores=NC, num_subcores=NS)
  run = pl.kernel(
      functools.partial(_ee_body, bpw, d, p),
      out_type=jax.ShapeDtypeStruct((b,), jnp.float32),
      mesh=mesh,
      compiler_params=pltpu.CompilerParams(needs_layout_passes=False),
      scratch_types=[
          pltpu.VMEM((bpw,), jnp.int32),
          pltpu.VMEM((bpw,), jnp.int32),
          pltpu.VMEM((bpw,), jnp.int32),
          pltpu.VMEM((bpw,), jnp.int32),
          pltpu.VMEM((CH, d * p), jnp.float32),
          pltpu.VMEM((CH, d * p), jnp.float32),
          pltpu.VMEM((CH, d * p), jnp.float32),
          pltpu.VMEM((CH, d * p), jnp.float32),
          pltpu.VMEM((bpw,), jnp.float32),
          pltpu.VMEM((bpw,), jnp.float32),
          pltpu.VMEM((L,), jnp.float32),
          pltpu.VMEM((bpw,), jnp.float32),
          pltpu.SemaphoreType.DMA,
          pltpu.SemaphoreType.DMA,
          pltpu.SemaphoreType.DMA,
      ],
  )
  return run(uidx, iidx, eu_pk, ei_pk,
             bias_user.astype(jnp.float32), bias_item.astype(jnp.float32),
             gm_arr)
